# Initial kernel scaffold; baseline (speedup 1.0000x reference)
import jax
import jax.numpy as jnp
from jax.experimental import pallas as pl


def kernel(preds, thresh, max_proposals):
    raise NotImplementedError("write your pallas kernel here")


# speedup vs baseline: 222.2824x; 222.2824x over previous
import functools

import jax
import jax.numpy as jnp
from jax import lax
from jax.experimental import pallas as pl
from jax.experimental.pallas import tpu as pltpu

_B = 128


def _suppress_body(thresh_ref, col_ref, row_ref, sup_ref, *, n_valid, nb):
    t = thresh_ref[0, 0]
    npad = nb * _B
    lane = lax.broadcasted_iota(jnp.int32, (1, npad), 1)
    sup_ref[...] = (lane >= n_valid).astype(jnp.float32)

    riota = lax.broadcasted_iota(jnp.int32, (_B, 1), 0)
    ciota = lax.broadcasted_iota(jnp.int32, (1, _B), 1)

    def _iou_mask(rows, cols):
        x1r = rows[:, 0:1]
        y1r = rows[:, 1:2]
        x2r = rows[:, 2:3]
        y2r = rows[:, 3:4]
        ar = rows[:, 4:5]
        x1c = cols[0:1, :]
        y1c = cols[1:2, :]
        x2c = cols[2:3, :]
        y2c = cols[3:4, :]
        ac = cols[4:5, :]
        iw = jnp.maximum(jnp.minimum(x2r, x2c) - jnp.maximum(x1r, x1c), 0.0)
        ih = jnp.maximum(jnp.minimum(y2r, y2c) - jnp.maximum(y1r, y1c), 0.0)
        inter = iw * ih
        iou = inter / ((ar + ac) - inter + 1e-9)
        return (iou > t).astype(jnp.float32)

    def outer(k, _):
        kb = k * _B
        rows = row_ref[pl.ds(kb, _B), :]
        cols_k = col_ref[:, pl.ds(kb, _B)]
        s_self = _iou_mask(rows, cols_k) * (riota < ciota).astype(jnp.float32)
        alive0 = 1.0 - sup_ref[0:1, pl.ds(kb, _B)]

        def wcond(c):
            return c[1]

        def wbody(c):
            a, _ = c
            s = jnp.dot(a, s_self, preferred_element_type=jnp.float32)
            a_new = alive0 * (s <= 0.0).astype(jnp.float32)
            return a_new, jnp.any(a_new != a)

        a, _ = lax.while_loop(wcond, wbody, (alive0, True))
        sup_ref[0:1, pl.ds(kb, _B)] = 1.0 - a

        def inner(j, _):
            jb = j * _B
            cols_j = col_ref[:, pl.ds(jb, _B)]
            m = _iou_mask(rows, cols_j)
            s = jnp.dot(a, m, preferred_element_type=jnp.float32)
            old = sup_ref[0:1, pl.ds(jb, _B)]
            sup_ref[0:1, pl.ds(jb, _B)] = jnp.maximum(old, (s > 0.0).astype(jnp.float32))
            return 0

        lax.fori_loop(k + 1, nb, inner, 0)
        return 0

    lax.fori_loop(0, nb, outer, 0)


def kernel(preds, thresh, max_proposals):
    n = preds.shape[0]
    k_out = 1000
    nb = -(-n // _B)
    npad = nb * _B

    scores = preds[:, 4]
    order = jnp.argsort(-scores)
    boxes = preds[order, :4]
    x1 = boxes[:, 0]
    y1 = boxes[:, 1]
    x2 = boxes[:, 2]
    y2 = boxes[:, 3]
    areas = jnp.maximum(x2 - x1, 0.0) * jnp.maximum(y2 - y1, 0.0)
    feat = jnp.stack([x1, y1, x2, y2, areas], axis=1)
    feat = jnp.pad(feat, ((0, npad - n), (0, 3)))

    sup = pl.pallas_call(
        functools.partial(_suppress_body, n_valid=n, nb=nb),
        out_shape=jax.ShapeDtypeStruct((1, npad), jnp.float32),
        in_specs=[
            pl.BlockSpec(memory_space=pltpu.SMEM),
            pl.BlockSpec(memory_space=pltpu.VMEM),
            pl.BlockSpec(memory_space=pltpu.VMEM),
        ],
        out_specs=pl.BlockSpec(memory_space=pltpu.VMEM),
    )(
        jnp.asarray(thresh, jnp.float32).reshape(1, 1),
        feat.T,
        feat,
    )

    alive = sup[0, :n] == 0.0
    rank = jnp.cumsum(alive) - 1
    idx = jnp.arange(n)
    pos = jnp.where(alive & (rank < max_proposals), idx, n)
    keep_sorted = jnp.sort(pos)[:k_out]
    keep = order[keep_sorted]
    return (preds[keep], keep)


# speedup vs baseline: 333.9638x; 1.5024x over previous
import functools

import jax
import jax.numpy as jnp
from jax import lax
from jax.experimental import pallas as pl
from jax.experimental.pallas import tpu as pltpu

_R = 64
_C = 128
_SZ = _R * _C
_LOG = 13
_PAD_IDX = 1 << 20


def _roll(x, shift):
    if shift % _C == 0:
        s = (shift // _C) % _R
        if s == 0:
            return x
        return jnp.concatenate([x[s:], x[:s]], axis=0)
    s = shift % _C
    return jnp.concatenate([x[:, s:], x[:, :s]], axis=1)


def _bitonic(arrs, before, r_io, c_io):
    for s in range(_LOG):
        if s + 1 >= 7:
            asc = ((r_io >> (s + 1 - 7)) & 1) == 0
        else:
            asc = ((c_io >> (s + 1)) & 1) == 0
        for sub in range(s, -1, -1):
            d = 1 << sub
            if d < _C:
                lower = (c_io & d) == 0
            else:
                lower = (r_io & (d // _C)) == 0
            ups = [_roll(a, d) for a in arrs]
            downs = [_roll(a, -d) for a in arrs]
            parts = [jnp.where(lower, u, dn) for u, dn in zip(ups, downs)]
            first = before(arrs, parts)
            take_self = first == (lower == asc)
            arrs = [jnp.where(take_self, a, p) for a, p in zip(arrs, parts)]
    return arrs


def _nms_body(thresh_ref, maxp_ref, conf_ref, idx_ref, x1_ref, y1_ref,
              x2_ref, y2_ref, keep_ref, x1s, y1s, x2s, y2s, ars, sups,
              *, n_valid, nb):
    t = thresh_ref[0, 0]
    maxp = maxp_ref[0, 0]
    r_io = lax.broadcasted_iota(jnp.int32, (_R, _C), 0)
    c_io = lax.broadcasted_iota(jnp.int32, (_R, _C), 1)

    def before1(a, b):
        return (a[0] > b[0]) | ((a[0] == b[0]) & (a[1] < b[1]))

    arrs = [conf_ref[...], idx_ref[...], x1_ref[...], y1_ref[...],
            x2_ref[...], y2_ref[...]]
    _, ord_val, x1v, y1v, x2v, y2v = _bitonic(arrs, before1, r_io, c_io)
    x1s[...] = x1v
    y1s[...] = y1v
    x2s[...] = x2v
    y2s[...] = y2v
    ars[...] = jnp.maximum(x2v - x1v, 0.0) * jnp.maximum(y2v - y1v, 0.0)
    flat = r_io * _C + c_io
    sups[...] = (flat >= n_valid).astype(jnp.float32)

    ident = (lax.broadcasted_iota(jnp.int32, (_C, _C), 0)
             == lax.broadcasted_iota(jnp.int32, (_C, _C), 1)).astype(jnp.float32)
    dn = (((1,), (1,)), ((), ()))

    def tcol(row):
        return lax.dot_general(ident, row, dn,
                               preferred_element_type=jnp.float32)

    ri = lax.broadcasted_iota(jnp.int32, (_C, 1), 0)
    ci = lax.broadcasted_iota(jnp.int32, (1, _C), 1)

    def iou_mask(rc, cj):
        x1r, y1r, x2r, y2r, ar = rc
        x1c = x1s[cj, :]
        y1c = y1s[cj, :]
        x2c = x2s[cj, :]
        y2c = y2s[cj, :]
        ac = ars[cj, :]
        iw = jnp.maximum(jnp.minimum(x2r, x2c) - jnp.maximum(x1r, x1c), 0.0)
        ih = jnp.maximum(jnp.minimum(y2r, y2c) - jnp.maximum(y1r, y1c), 0.0)
        inter = iw * ih
        iou = inter / ((ar + ac) - inter + 1e-9)
        return (iou > t).astype(jnp.float32)

    def outer(k, _):
        kk = pl.ds(k, 1)
        rc = (tcol(x1s[kk, :]), tcol(y1s[kk, :]), tcol(x2s[kk, :]),
              tcol(y2s[kk, :]), tcol(ars[kk, :]))
        s_self = iou_mask(rc, kk) * (ri < ci).astype(jnp.float32)
        alive0 = 1.0 - sups[kk, :]

        def wbody(c):
            a, _ = c
            s = lax.dot_general(a, s_self, dn2,
                                preferred_element_type=jnp.float32)
            a_new = alive0 * (s <= 0.0).astype(jnp.float32)
            return a_new, jnp.any(a_new != a)

        dn2 = (((1,), (0,)), ((), ()))
        a, _ = lax.while_loop(lambda c: c[1], wbody, (alive0, True))
        sups[kk, :] = 1.0 - a

        def inner(j, _):
            jj = pl.ds(j, 1)
            m = iou_mask(rc, jj)
            s = lax.dot_general(a, m, dn2, preferred_element_type=jnp.float32)
            sups[jj, :] = jnp.maximum(sups[jj, :],
                                      (s > 0.0).astype(jnp.float32))
            return 0

        lax.fori_loop(k + 1, nb, inner, 0)
        return 0

    lax.fori_loop(0, nb, outer, 0)

    alive = 1.0 - sups[...]
    lt = (lax.broadcasted_iota(jnp.int32, (_C, _C), 0)
          <= lax.broadcasted_iota(jnp.int32, (_C, _C), 1)).astype(jnp.float32)
    cum = jnp.dot(alive, lt, preferred_element_type=jnp.float32)
    rowsum = jnp.sum(alive, axis=1, keepdims=True)
    sl = (lax.broadcasted_iota(jnp.int32, (_R, _R), 0)
          > lax.broadcasted_iota(jnp.int32, (_R, _R), 1)).astype(jnp.float32)
    offs = jnp.dot(sl, rowsum, preferred_element_type=jnp.float32)
    rank = cum + offs - 1.0

    sel = (alive > 0.0) & (rank < maxp.astype(jnp.float32))
    key2 = jnp.where(sel, flat, _SZ + flat)
    last = lax.slice(ord_val, ((n_valid - 1) // _C, (n_valid - 1) % _C),
                     ((n_valid - 1) // _C + 1, (n_valid - 1) % _C + 1))
    pay2 = jnp.where(sel, ord_val, last)

    def before2(a, b):
        return a[0] < b[0]

    _, keep_sorted = _bitonic([key2, pay2], before2, r_io, c_io)
    keep_ref[...] = keep_sorted[0:8, :]


def kernel(preds, thresh, max_proposals):
    n = preds.shape[0]
    k_out = 1000
    nb = -(-n // _C)

    def fpad(col, fill):
        return jnp.pad(col, (0, _SZ - n), constant_values=fill).reshape(_R, _C)

    conf0 = fpad(preds[:, 4], -1.0)
    idx0 = fpad(jnp.arange(n, dtype=jnp.int32), _PAD_IDX)
    x10 = fpad(preds[:, 0], 0.0)
    y10 = fpad(preds[:, 1], 0.0)
    x20 = fpad(preds[:, 2], 0.0)
    y20 = fpad(preds[:, 3], 0.0)

    keep_mat = pl.pallas_call(
        functools.partial(_nms_body, n_valid=n, nb=nb),
        out_shape=jax.ShapeDtypeStruct((8, _C), jnp.int32),
        in_specs=[
            pl.BlockSpec(memory_space=pltpu.SMEM),
            pl.BlockSpec(memory_space=pltpu.SMEM),
        ] + [pl.BlockSpec(memory_space=pltpu.VMEM)] * 6,
        out_specs=pl.BlockSpec(memory_space=pltpu.VMEM),
        scratch_shapes=[pltpu.VMEM((_R, _C), jnp.float32)] * 6,
    )(
        jnp.asarray(thresh, jnp.float32).reshape(1, 1),
        jnp.asarray(max_proposals, jnp.int32).reshape(1, 1),
        conf0, idx0, x10, y10, x20, y20,
    )

    keep = keep_mat.reshape(-1)[:k_out]
    return (preds[keep], keep)


# speedup vs baseline: 1260.2328x; 3.7736x over previous
import functools

import jax
import jax.numpy as jnp
from jax import lax
from jax.experimental import pallas as pl
from jax.experimental.pallas import tpu as pltpu

_R = 64
_C = 128
_SZ = _R * _C
_LOG = 13
_PAD_IDX = 1 << 20


def _roll(x, shift):
    if shift % _C == 0:
        s = (shift // _C) % _R
        if s == 0:
            return x
        return jnp.concatenate([x[s:], x[:s]], axis=0)
    s = shift % _C
    return jnp.concatenate([x[:, s:], x[:, :s]], axis=1)


def _bitonic(arrs, before, r_io, c_io):
    for s in range(_LOG):
        if s + 1 >= 7:
            asc = ((r_io >> (s + 1 - 7)) & 1) == 0
        else:
            asc = ((c_io >> (s + 1)) & 1) == 0
        for sub in range(s, -1, -1):
            d = 1 << sub
            if d < _C:
                lower = (c_io & d) == 0
            else:
                lower = (r_io & (d // _C)) == 0
            ups = [_roll(a, d) for a in arrs]
            downs = [_roll(a, -d) for a in arrs]
            parts = [jnp.where(lower, u, dn) for u, dn in zip(ups, downs)]
            first = before(arrs, parts)
            take_self = first == (lower == asc)
            arrs = [jnp.where(take_self, a, p) for a, p in zip(arrs, parts)]
    return arrs


def _nms_body(thresh_ref, maxp_ref, conf_ref, idx_ref, x1_ref, y1_ref,
              x2_ref, y2_ref, keep_ref, x1s, y1s, x2s, y2s, ars, sups,
              *, n_valid, nb):
    t = thresh_ref[0, 0]
    maxp = maxp_ref[0, 0]
    r_io = lax.broadcasted_iota(jnp.int32, (_R, _C), 0)
    c_io = lax.broadcasted_iota(jnp.int32, (_R, _C), 1)

    def before1(a, b):
        return (a[0] > b[0]) | ((a[0] == b[0]) & (a[1] < b[1]))

    arrs = [conf_ref[...], idx_ref[...], x1_ref[...], y1_ref[...],
            x2_ref[...], y2_ref[...]]
    _, ord_val, x1v, y1v, x2v, y2v = _bitonic(arrs, before1, r_io, c_io)
    x1s[...] = x1v
    y1s[...] = y1v
    x2s[...] = x2v
    y2s[...] = y2v
    ars[...] = jnp.maximum(x2v - x1v, 0.0) * jnp.maximum(y2v - y1v, 0.0)
    flat = r_io * _C + c_io
    sups[...] = (flat >= n_valid).astype(jnp.float32)

    ident = (lax.broadcasted_iota(jnp.int32, (_C, _C), 0)
             == lax.broadcasted_iota(jnp.int32, (_C, _C), 1)).astype(jnp.float32)
    dn = (((1,), (1,)), ((), ()))

    def tcol(row):
        return lax.dot_general(ident, row, dn,
                               preferred_element_type=jnp.float32)

    ri = lax.broadcasted_iota(jnp.int32, (_C, 1), 0)
    ci = lax.broadcasted_iota(jnp.int32, (1, _C), 1)

    def iou_mask(rc, cj):
        x1r, y1r, x2r, y2r, ar = rc
        x1c = x1s[cj, :]
        y1c = y1s[cj, :]
        x2c = x2s[cj, :]
        y2c = y2s[cj, :]
        ac = ars[cj, :]
        iw = jnp.maximum(jnp.minimum(x2r, x2c) - jnp.maximum(x1r, x1c), 0.0)
        ih = jnp.maximum(jnp.minimum(y2r, y2c) - jnp.maximum(y1r, y1c), 0.0)
        inter = iw * ih
        iou = inter / ((ar + ac) - inter + 1e-9)
        return (iou > t).astype(jnp.float32)

    dn2 = (((1,), (0,)), ((), ()))
    maxp_f = maxp.astype(jnp.float32)

    def tcols(bb):
        return (tcol(x1s[bb, :]), tcol(y1s[bb, :]), tcol(x2s[bb, :]),
                tcol(y2s[bb, :]), tcol(ars[bb, :]))

    def jbody(carry):
        j, count = carry
        jj = pl.ds(j, 1)

        def ibody(i, s_or):
            ii = pl.ds(i, 1)
            m = iou_mask(tcols(ii), jj)
            a_i = 1.0 - sups[ii, :]
            s = lax.dot_general(a_i, m, dn2,
                                preferred_element_type=jnp.float32)
            return jnp.maximum(s_or, (s > 0.0).astype(jnp.float32))

        s_or = lax.fori_loop(0, j, ibody, jnp.zeros((1, _C), jnp.float32))
        alive0 = 1.0 - jnp.maximum(sups[jj, :], s_or)
        s_self = iou_mask(tcols(jj), jj) * (ri < ci).astype(jnp.float32)

        def wbody(c):
            a, _ = c
            s = lax.dot_general(a, s_self, dn2,
                                preferred_element_type=jnp.float32)
            a_new = alive0 * (s <= 0.0).astype(jnp.float32)
            return a_new, jnp.any(a_new != a)

        a, _ = lax.while_loop(lambda c: c[1], wbody, (alive0, True))
        sups[jj, :] = 1.0 - a
        return j + 1, count + jnp.sum(a)

    def jcond(carry):
        j, count = carry
        return (j < nb) & (count < maxp_f)

    lax.while_loop(jcond, jbody, (0, 0.0))

    alive = 1.0 - sups[...]
    lt = (lax.broadcasted_iota(jnp.int32, (_C, _C), 0)
          <= lax.broadcasted_iota(jnp.int32, (_C, _C), 1)).astype(jnp.float32)
    cum = jnp.dot(alive, lt, preferred_element_type=jnp.float32)
    rowsum = jnp.sum(alive, axis=1, keepdims=True)
    sl = (lax.broadcasted_iota(jnp.int32, (_R, _R), 0)
          > lax.broadcasted_iota(jnp.int32, (_R, _R), 1)).astype(jnp.float32)
    offs = jnp.dot(sl, rowsum, preferred_element_type=jnp.float32)
    rank = cum + offs - 1.0

    sel = (alive > 0.0) & (rank < maxp.astype(jnp.float32))
    key2 = jnp.where(sel, flat, _SZ + flat)
    last = lax.slice(ord_val, ((n_valid - 1) // _C, (n_valid - 1) % _C),
                     ((n_valid - 1) // _C + 1, (n_valid - 1) % _C + 1))
    pay2 = jnp.where(sel, ord_val, last)

    def before2(a, b):
        return a[0] < b[0]

    _, keep_sorted = _bitonic([key2, pay2], before2, r_io, c_io)
    keep_ref[...] = keep_sorted[0:8, :]


def kernel(preds, thresh, max_proposals):
    n = preds.shape[0]
    k_out = 1000
    nb = -(-n // _C)

    def fpad(col, fill):
        return jnp.pad(col, (0, _SZ - n), constant_values=fill).reshape(_R, _C)

    conf0 = fpad(preds[:, 4], -1.0)
    idx0 = fpad(jnp.arange(n, dtype=jnp.int32), _PAD_IDX)
    x10 = fpad(preds[:, 0], 0.0)
    y10 = fpad(preds[:, 1], 0.0)
    x20 = fpad(preds[:, 2], 0.0)
    y20 = fpad(preds[:, 3], 0.0)

    keep_mat = pl.pallas_call(
        functools.partial(_nms_body, n_valid=n, nb=nb),
        out_shape=jax.ShapeDtypeStruct((8, _C), jnp.int32),
        in_specs=[
            pl.BlockSpec(memory_space=pltpu.SMEM),
            pl.BlockSpec(memory_space=pltpu.SMEM),
        ] + [pl.BlockSpec(memory_space=pltpu.VMEM)] * 6,
        out_specs=pl.BlockSpec(memory_space=pltpu.VMEM),
        scratch_shapes=[pltpu.VMEM((_R, _C), jnp.float32)] * 6,
    )(
        jnp.asarray(thresh, jnp.float32).reshape(1, 1),
        jnp.asarray(max_proposals, jnp.int32).reshape(1, 1),
        conf0, idx0, x10, y10, x20, y20,
    )

    keep = keep_mat.reshape(-1)[:k_out]
    return (preds[keep], keep)
